# baseline (speedup 1.0000x reference)
import functools

import jax
import jax.numpy as jnp
from jax import lax
from jax.experimental import pallas as pl
from jax.experimental.pallas import tpu as pltpu
from jax.experimental.pallas import tpu_sc as plsc

_B = 16384
_NNEG = 20
_D = 128
_C2 = 64
_CN = 128
_NBUF = 5


def _run_segment(nchunks, nbuf, start, wait_gather, start_stores,
                 wait_stores):
  look = nbuf - 1
  nblocks = nchunks // nbuf

  for g in range(min(look, nchunks)):
    start(g, g % nbuf)

  def position_full(j, b):
    bg = (b + look) % nbuf
    wait_stores(j + look - nbuf, bg)
    start(j + look, bg)
    wait_gather(j, b)
    start_stores(j, b)

  for b in range(min(nbuf, nchunks)):
    j = b
    g = j + look
    if g < nchunks:
      bg = g % nbuf
      if g >= nbuf:
        wait_stores(g - nbuf, bg)
      start(g, bg)
    wait_gather(j, b)
    start_stores(j, b)

  if nblocks >= 3:
    def body(i, carry):
      j0 = i * nbuf
      for b in range(nbuf):
        position_full(j0 + b, b)
      return carry
    lax.fori_loop(1, nblocks - 1, body, 0)

  if nblocks >= 2:
    j0 = (nblocks - 1) * nbuf
    for b in range(nbuf):
      j = j0 + b
      g = j + look
      if g < nchunks:
        bg = g % nbuf
        wait_stores(g - nbuf, bg)
        start(g, bg)
      wait_gather(j, b)
      start_stores(j, b)

  for j in range(max(0, nchunks - nbuf), nchunks):
    wait_stores(j, j % nbuf)


def _seg(table, idx_v, out, out_base, nrows, chunk, bufs, gsems, ssems):
  nchunks = nrows // chunk

  def start(j, b):
    pltpu.async_copy(table.at[idx_v.at[pl.ds(j * chunk, chunk)]],
                     bufs[b].at[pl.ds(0, chunk)], gsems[b])

  def wait_gather(j, b):
    pltpu.make_async_copy(table.at[idx_v.at[pl.ds(j * chunk, chunk)]],
                          bufs[b].at[pl.ds(0, chunk)], gsems[b]).wait()

  def start_stores(j, b):
    pltpu.async_copy(bufs[b].at[pl.ds(0, chunk)],
                     out.at[pl.ds(out_base + j * chunk, chunk)], ssems[b])

  def wait_stores(j, b):
    pltpu.make_async_copy(bufs[b].at[pl.ds(0, chunk)],
                          out.at[pl.ds(out_base + j * chunk, chunk)],
                          ssems[b]).wait()

  _run_segment(nchunks, len(bufs), start, wait_gather, start_stores,
               wait_stores)


def _make_sc_gather():
  info = plsc.get_sparse_core_info()
  nc, ns = info.num_cores, info.num_subcores
  nw = nc * ns
  bt = _B // nw
  bn = (_B * _NNEG) // nw
  mesh = plsc.VectorSubcoreMesh(core_axis_name="c", subcore_axis_name="s")

  @functools.partial(
      pl.kernel,
      mesh=mesh,
      out_type=(
          jax.ShapeDtypeStruct((_B, _D), jnp.float32),
          jax.ShapeDtypeStruct((_B, _D), jnp.float32),
          jax.ShapeDtypeStruct((_B * _NNEG, _D), jnp.float32),
      ),
      scratch_types=[
          pltpu.VMEM((bt,), jnp.int32),
          pltpu.VMEM((bt,), jnp.int32),
          pltpu.VMEM((bn,), jnp.int32),
      ] + [pltpu.VMEM((_CN, _D), jnp.float32) for _ in range(_NBUF)]
        + [pltpu.SemaphoreType.DMA for _ in range(2 * _NBUF)],
      compiler_params=pltpu.CompilerParams(use_tc_tiling_on_sc=True),
  )
  def sc_gather(tgt_hbm, ctx_hbm, noise_hbm, in_emb, out_emb,
                out_t, out_c, out_n,
                idx_t, idx_c, idx_n,
                b0, b1, b2, b3, b4,
                g0, g1, g2, g3, g4, s0, s1, s2, s3, s4):
    wid = lax.axis_index("s") * nc + lax.axis_index("c")
    pltpu.sync_copy(tgt_hbm.at[pl.ds(wid * bt, bt)], idx_t)
    pltpu.sync_copy(ctx_hbm.at[pl.ds(wid * bt, bt)], idx_c)
    pltpu.sync_copy(noise_hbm.at[pl.ds(wid * bn, bn)], idx_n)
    bufs = (b0, b1, b2, b3, b4)
    gsems = (g0, g1, g2, g3, g4)
    ssems = (s0, s1, s2, s3, s4)
    _seg(in_emb, idx_t, out_t, wid * bt, bt, _C2, bufs[:4], gsems[:4],
         ssems[:4])
    _seg(out_emb, idx_c, out_c, wid * bt, bt, _C2, bufs[:4], gsems[:4],
         ssems[:4])
    _seg(out_emb, idx_n, out_n, wid * bn, bn, _CN, bufs, gsems, ssems)

  return sc_gather


_sc_gather = _make_sc_gather()


def kernel(target, context, noise, input_embedding, output_embedding):
  noise_t = jnp.transpose(noise.astype(jnp.int32)).reshape(-1)
  out_t, out_c, out_n = _sc_gather(
      target.astype(jnp.int32),
      context.astype(jnp.int32),
      noise_t,
      input_embedding,
      output_embedding,
  )
  return (out_t, out_c,
          jnp.transpose(out_n.reshape(_NNEG, _B, _D), (1, 0, 2)))


# speedup vs baseline: 1.0012x; 1.0012x over previous
import functools

import jax
import jax.numpy as jnp
from jax import lax
from jax.experimental import pallas as pl
from jax.experimental.pallas import tpu as pltpu
from jax.experimental.pallas import tpu_sc as plsc

_B = 16384
_NNEG = 20
_D = 128
_C2 = 64
_CN = 64
_NBUF = 10


def _run_segment(nchunks, nbuf, start, wait_gather, start_stores,
                 wait_stores):
  look = nbuf - 1
  nblocks = nchunks // nbuf

  for g in range(min(look, nchunks)):
    start(g, g % nbuf)

  def position_full(j, b):
    bg = (b + look) % nbuf
    wait_stores(j + look - nbuf, bg)
    start(j + look, bg)
    wait_gather(j, b)
    start_stores(j, b)

  for b in range(min(nbuf, nchunks)):
    j = b
    g = j + look
    if g < nchunks:
      bg = g % nbuf
      if g >= nbuf:
        wait_stores(g - nbuf, bg)
      start(g, bg)
    wait_gather(j, b)
    start_stores(j, b)

  if nblocks >= 3:
    def body(i, carry):
      j0 = i * nbuf
      for b in range(nbuf):
        position_full(j0 + b, b)
      return carry
    lax.fori_loop(1, nblocks - 1, body, 0)

  if nblocks >= 2:
    j0 = (nblocks - 1) * nbuf
    for b in range(nbuf):
      j = j0 + b
      g = j + look
      if g < nchunks:
        bg = g % nbuf
        wait_stores(g - nbuf, bg)
        start(g, bg)
      wait_gather(j, b)
      start_stores(j, b)

  for j in range(max(0, nchunks - nbuf), nchunks):
    wait_stores(j, j % nbuf)


def _seg(table, idx_v, out, out_base, nrows, chunk, bufs, gsems, ssems):
  nchunks = nrows // chunk

  def start(j, b):
    pltpu.async_copy(table.at[idx_v.at[pl.ds(j * chunk, chunk)]],
                     bufs[b].at[pl.ds(0, chunk)], gsems[b])

  def wait_gather(j, b):
    pltpu.make_async_copy(table.at[idx_v.at[pl.ds(j * chunk, chunk)]],
                          bufs[b].at[pl.ds(0, chunk)], gsems[b]).wait()

  def start_stores(j, b):
    pltpu.async_copy(bufs[b].at[pl.ds(0, chunk)],
                     out.at[pl.ds(out_base + j * chunk, chunk)], ssems[b])

  def wait_stores(j, b):
    pltpu.make_async_copy(bufs[b].at[pl.ds(0, chunk)],
                          out.at[pl.ds(out_base + j * chunk, chunk)],
                          ssems[b]).wait()

  _run_segment(nchunks, len(bufs), start, wait_gather, start_stores,
               wait_stores)


def _make_sc_gather():
  info = plsc.get_sparse_core_info()
  nc, ns = info.num_cores, info.num_subcores
  nw = nc * ns
  bt = _B // nw
  bn = (_B * _NNEG) // nw
  mesh = plsc.VectorSubcoreMesh(core_axis_name="c", subcore_axis_name="s")

  @functools.partial(
      pl.kernel,
      mesh=mesh,
      out_type=(
          jax.ShapeDtypeStruct((_B, _D), jnp.float32),
          jax.ShapeDtypeStruct((_B, _D), jnp.float32),
          jax.ShapeDtypeStruct((_B * _NNEG, _D), jnp.float32),
      ),
      scratch_types=[
          pltpu.VMEM((bt,), jnp.int32),
          pltpu.VMEM((bt,), jnp.int32),
          pltpu.VMEM((bn,), jnp.int32),
      ] + [pltpu.VMEM((_CN, _D), jnp.float32) for _ in range(_NBUF)]
        + [pltpu.SemaphoreType.DMA for _ in range(2 * _NBUF)],
      compiler_params=pltpu.CompilerParams(use_tc_tiling_on_sc=True),
  )
  def sc_gather(tgt_hbm, ctx_hbm, noise_hbm, in_emb, out_emb,
                out_t, out_c, out_n,
                idx_t, idx_c, idx_n,
                b0, b1, b2, b3, b4, b5, b6, b7, b8, b9,
                g0, g1, g2, g3, g4, g5, g6, g7, g8, g9,
                s0, s1, s2, s3, s4, s5, s6, s7, s8, s9):
    wid = lax.axis_index("s") * nc + lax.axis_index("c")
    pltpu.sync_copy(tgt_hbm.at[pl.ds(wid * bt, bt)], idx_t)
    pltpu.sync_copy(ctx_hbm.at[pl.ds(wid * bt, bt)], idx_c)
    pltpu.sync_copy(noise_hbm.at[pl.ds(wid * bn, bn)], idx_n)
    bufs = (b0, b1, b2, b3, b4, b5, b6, b7, b8, b9)
    gsems = (g0, g1, g2, g3, g4, g5, g6, g7, g8, g9)
    ssems = (s0, s1, s2, s3, s4, s5, s6, s7, s8, s9)
    _seg(in_emb, idx_t, out_t, wid * bt, bt, _C2, bufs[:4], gsems[:4],
         ssems[:4])
    _seg(out_emb, idx_c, out_c, wid * bt, bt, _C2, bufs[:4], gsems[:4],
         ssems[:4])
    _seg(out_emb, idx_n, out_n, wid * bn, bn, _CN, bufs, gsems, ssems)

  return sc_gather


_sc_gather = _make_sc_gather()


def kernel(target, context, noise, input_embedding, output_embedding):
  noise_t = jnp.transpose(noise.astype(jnp.int32)).reshape(-1)
  out_t, out_c, out_n = _sc_gather(
      target.astype(jnp.int32),
      context.astype(jnp.int32),
      noise_t,
      input_embedding,
      output_embedding,
  )
  return (out_t, out_c,
          jnp.transpose(out_n.reshape(_NNEG, _B, _D), (1, 0, 2)))


# speedup vs baseline: 1.0098x; 1.0086x over previous
import functools

import jax
import jax.numpy as jnp
from jax import lax
from jax.experimental import pallas as pl
from jax.experimental.pallas import tpu as pltpu
from jax.experimental.pallas import tpu_sc as plsc

_B = 16384
_NNEG = 20
_D = 128
_C2 = 64
_CN = 64
_NBUF = 10


def _run_segment(nchunks, nbuf, start, wait_gather, start_stores,
                 wait_stores):
  look = nbuf - 1
  nblocks = nchunks // nbuf

  for g in range(min(look, nchunks)):
    start(g, g % nbuf)

  def position_full(j, b):
    bg = (b + look) % nbuf
    wait_stores(j + look - nbuf, bg)
    start(j + look, bg)
    wait_gather(j, b)
    start_stores(j, b)

  for b in range(min(nbuf, nchunks)):
    j = b
    g = j + look
    if g < nchunks:
      bg = g % nbuf
      if g >= nbuf:
        wait_stores(g - nbuf, bg)
      start(g, bg)
    wait_gather(j, b)
    start_stores(j, b)

  if nblocks >= 3:
    def body(i, carry):
      j0 = i * nbuf
      for b in range(nbuf):
        position_full(j0 + b, b)
      return carry
    lax.fori_loop(1, nblocks - 1, body, 0)

  if nblocks >= 2:
    j0 = (nblocks - 1) * nbuf
    for b in range(nbuf):
      j = j0 + b
      g = j + look
      if g < nchunks:
        bg = g % nbuf
        wait_stores(g - nbuf, bg)
        start(g, bg)
      wait_gather(j, b)
      start_stores(j, b)

  for j in range(max(0, nchunks - nbuf), nchunks):
    wait_stores(j, j % nbuf)


def _seg(table, idx_v, out, out_base, nrows, chunk, bufs, gsems, ssems):
  nchunks = nrows // chunk

  def start(j, b):
    pltpu.async_copy(table.at[idx_v.at[pl.ds(j * chunk, chunk)]],
                     bufs[b].at[pl.ds(0, chunk)], gsems[b])

  def wait_gather(j, b):
    pltpu.make_async_copy(table.at[idx_v.at[pl.ds(j * chunk, chunk)]],
                          bufs[b].at[pl.ds(0, chunk)], gsems[b]).wait()

  def start_stores(j, b):
    pltpu.async_copy(bufs[b].at[pl.ds(0, chunk)],
                     out.at[pl.ds(out_base + j * chunk, chunk)], ssems[b])

  def wait_stores(j, b):
    pltpu.make_async_copy(bufs[b].at[pl.ds(0, chunk)],
                          out.at[pl.ds(out_base + j * chunk, chunk)],
                          ssems[b]).wait()

  _run_segment(nchunks, len(bufs), start, wait_gather, start_stores,
               wait_stores)


def _make_sc_gather():
  info = plsc.get_sparse_core_info()
  nc, ns = info.num_cores, info.num_subcores
  nw = nc * ns
  bt = _B // nw
  bn = (_B * _NNEG) // nw
  mesh = plsc.VectorSubcoreMesh(core_axis_name="c", subcore_axis_name="s")

  @functools.partial(
      pl.kernel,
      mesh=mesh,
      out_type=(
          jax.ShapeDtypeStruct((_B, _D), jnp.float32),
          jax.ShapeDtypeStruct((_B, _D), jnp.float32),
          jax.ShapeDtypeStruct((_B * _NNEG, _D), jnp.float32),
      ),
      scratch_types=[
          pltpu.VMEM((bt,), jnp.int32),
          pltpu.VMEM((bt,), jnp.int32),
          pltpu.VMEM((bn,), jnp.int32),
      ] + [pltpu.VMEM((_CN, _D), jnp.float32) for _ in range(_NBUF)]
        + [pltpu.SemaphoreType.DMA for _ in range(2 * _NBUF)],
      compiler_params=pltpu.CompilerParams(use_tc_tiling_on_sc=True),
  )
  def sc_gather(tgt_hbm, ctx_hbm, noise_hbm, in_emb, out_emb,
                out_t, out_c, out_n,
                idx_t, idx_c, idx_n,
                b0, b1, b2, b3, b4, b5, b6, b7, b8, b9,
                g0, g1, g2, g3, g4, g5, g6, g7, g8, g9,
                s0, s1, s2, s3, s4, s5, s6, s7, s8, s9):
    wid = lax.axis_index("s") * nc + lax.axis_index("c")
    bufs = (b0, b1, b2, b3, b4, b5, b6, b7, b8, b9)
    gsems = (g0, g1, g2, g3, g4, g5, g6, g7, g8, g9)
    ssems = (s0, s1, s2, s3, s4, s5, s6, s7, s8, s9)
    pltpu.sync_copy(tgt_hbm.at[pl.ds(wid * bt, bt)], idx_t)
    pltpu.async_copy(ctx_hbm.at[pl.ds(wid * bt, bt)], idx_c, ssems[4])
    pltpu.async_copy(noise_hbm.at[pl.ds(wid * bn, bn)], idx_n, ssems[5])
    _seg(in_emb, idx_t, out_t, wid * bt, bt, _C2, bufs[:4], gsems[:4],
         ssems[:4])
    pltpu.make_async_copy(ctx_hbm.at[pl.ds(wid * bt, bt)], idx_c,
                          ssems[4]).wait()
    _seg(out_emb, idx_c, out_c, wid * bt, bt, _C2, bufs[:4], gsems[:4],
         ssems[:4])
    pltpu.make_async_copy(noise_hbm.at[pl.ds(wid * bn, bn)], idx_n,
                          ssems[5]).wait()
    _seg(out_emb, idx_n, out_n, wid * bn, bn, _CN, bufs, gsems, ssems)

  return sc_gather


_sc_gather = _make_sc_gather()


def kernel(target, context, noise, input_embedding, output_embedding):
  noise_t = jnp.transpose(noise.astype(jnp.int32)).reshape(-1)
  out_t, out_c, out_n = _sc_gather(
      target.astype(jnp.int32),
      context.astype(jnp.int32),
      noise_t,
      input_embedding,
      output_embedding,
  )
  return (out_t, out_c,
          jnp.transpose(out_n.reshape(_NNEG, _B, _D), (1, 0, 2)))
